# baseline (speedup 1.0000x reference)
import functools

import jax
import jax.numpy as jnp
from jax import lax
from jax.experimental import pallas as pl
from jax.experimental.pallas import tpu as pltpu
from jax.experimental.pallas import tpu_sc as plsc

_NUM_CORES = 2
_NUM_SUBCORES = 16
_NW = _NUM_CORES * _NUM_SUBCORES

_LANES = 128
_SUBS = 8
_TILE_WORDS = _SUBS * _LANES
_CHUNK_TILES = 128


@functools.partial(jax.jit, static_argnums=(2, 3))
def _gather2(x, table_t, B, D):
    b_per_w = B // _NW
    V = table_t.shape[1]
    n_tc = (V + _LANES - 1) // _LANES
    n_tr = D // _SUBS
    n_full = n_tc // _CHUNK_TILES
    rem_tiles = n_tc - n_full * _CHUNK_TILES
    n_ck = n_full + 1
    full_total = n_tr * n_full
    per_w = (full_total + _NW - 1) // _NW
    chunk_lanes = _CHUNK_TILES * _LANES
    flat_rows = n_tr * n_ck * _SUBS
    tr_stride_flat = n_ck * _CHUNK_TILES * _TILE_WORDS
    flat_len = flat_rows * chunk_lanes
    mesh = plsc.VectorSubcoreMesh(core_axis_name="c", subcore_axis_name="s")

    @functools.partial(
        pl.kernel,
        out_type=jax.ShapeDtypeStruct((flat_rows, chunk_lanes), jnp.float32),
        mesh=mesh,
        scratch_types=[pltpu.SemaphoreType.DMA],
        compiler_params=pltpu.CompilerParams(disable_bounds_checks=True),
    )
    def ka(table_hbm, flat_hbm, sem):
        wid = lax.axis_index("s") * _NUM_CORES + lax.axis_index("c")
        copies = []
        for k in range(per_w):
            cf = wid * per_w + k
            live = cf < full_total
            tr = cf // n_full
            cc = cf % n_full
            src = table_hbm.at[
                pl.ds(pl.multiple_of(tr * _SUBS, _SUBS), _SUBS),
                pl.ds(pl.multiple_of(cc * chunk_lanes, _LANES), chunk_lanes),
            ]
            dst = flat_hbm.at[
                pl.ds(
                    pl.multiple_of((tr * n_ck + cc) * _SUBS, _SUBS), _SUBS
                ),
                :,
            ]
            d = pltpu.make_async_copy(src, dst, sem)
            pl.when(live)(d.start)
            copies.append((live, d))
        rem_live = wid < n_tr
        rem_src = table_hbm.at[
            pl.ds(pl.multiple_of(wid * _SUBS, _SUBS), _SUBS),
            pl.ds(
                pl.multiple_of(n_full * chunk_lanes, _LANES),
                rem_tiles * _LANES,
            ),
        ]
        rem_dst = flat_hbm.at[
            pl.ds(
                pl.multiple_of((wid * n_ck + n_full) * _SUBS, _SUBS), _SUBS
            ),
            pl.ds(0, rem_tiles * _LANES),
        ]
        rd = pltpu.make_async_copy(rem_src, rem_dst, sem)
        pl.when(rem_live)(rd.start)
        for live, d in copies:
            pl.when(live)(d.wait)
        pl.when(rem_live)(rd.wait)

    @functools.partial(
        pl.kernel,
        out_type=jax.ShapeDtypeStruct((D, B), jnp.float32),
        mesh=mesh,
        scratch_types=[
            pltpu.VMEM((b_per_w,), jnp.int32),
            pltpu.VMEM((D, b_per_w), jnp.int32),
            pltpu.VMEM((D, b_per_w), jnp.float32),
            pltpu.SemaphoreType.DMA,
        ],
        compiler_params=pltpu.CompilerParams(use_tc_tiling_on_sc=False),
    )
    def kb(x_hbm, flat_hbm, out_t_hbm, idx_v, offs_v, rows_v, sem):
        wid = lax.axis_index("s") * _NUM_CORES + lax.axis_index("c")
        base = wid * b_per_w
        pltpu.sync_copy(x_hbm.at[pl.ds(base, b_per_w)], idx_v)
        n16 = b_per_w // 16

        def obody(j, carry):
            v = idx_v[pl.ds(j * 16, 16)]
            b0 = (v >> 14) * (_SUBS * chunk_lanes) + (v & (chunk_lanes - 1))
            for d in range(D):
                doff = (
                    (d // _SUBS) * tr_stride_flat
                    + (d % _SUBS) * chunk_lanes
                )
                offs_v[d, pl.ds(j * 16, 16)] = b0 + doff
            return carry

        lax.fori_loop(0, n16, obody, 0)
        copies = [
            pltpu.async_copy(flat_hbm.at[offs_v.at[d]], rows_v.at[d], sem)
            for d in range(D)
        ]
        for c in copies:
            c.wait()
        pltpu.sync_copy(rows_v, out_t_hbm.at[:, pl.ds(base, b_per_w)])

    flat = ka(table_t).reshape(flat_len)
    return kb(x, flat)


def kernel(x, table):
    (B,) = x.shape
    D = table.shape[1]
    out_t = _gather2(x.astype(jnp.int32), table.T, B, D)
    return out_t.T
